# Initial kernel scaffold; baseline (speedup 1.0000x reference)
import jax
import jax.numpy as jnp
from jax.experimental import pallas as pl


def kernel(decoder_states, encoder_states, step, input_lengths, provided_attention):
    raise NotImplementedError("write your pallas kernel here")


# speedup vs baseline: 35.9025x; 35.9025x over previous
import jax
import jax.numpy as jnp
from jax.experimental import pallas as pl
from jax.experimental.pallas import tpu as pltpu

_D_TILE = 256


def _onehot_kernel(idx_ref, out_ref):
    idx = idx_ref[0]
    e = jax.lax.broadcasted_iota(jnp.int32, out_ref.shape, 1)
    out_ref[...] = jnp.where(e == idx.reshape(_D_TILE, 1), 1.0, -jnp.inf)


def kernel(decoder_states, encoder_states, step, input_lengths, provided_attention):
    B, dec_seqlen = provided_attention.shape
    enc_seqlen = encoder_states.shape[1]
    rows = B * dec_seqlen
    n_tiles = rows // _D_TILE
    idx = jnp.asarray(provided_attention, jnp.int32).reshape(n_tiles, 1, _D_TILE)

    out = pl.pallas_call(
        _onehot_kernel,
        grid=(n_tiles,),
        in_specs=[pl.BlockSpec((1, 1, _D_TILE), lambda i: (i, 0, 0))],
        out_specs=pl.BlockSpec((_D_TILE, enc_seqlen), lambda i: (i, 0)),
        out_shape=jax.ShapeDtypeStruct((rows, enc_seqlen), jnp.float32),
    )(idx)
    return out.reshape(B, dec_seqlen, enc_seqlen)
